# baseline (speedup 1.0000x reference)
import functools

import jax
import jax.numpy as jnp
from jax import lax
from jax.experimental import pallas as pl
from jax.experimental.pallas import tpu as pltpu
from jax.experimental.pallas import tpu_sc as plsc

B = 1024
F = 2048
C = 100000
D = 512
K = 10
CC = 1024
NCH = 98
CPAD = NCH * CC
KSLOT = 16

NEG = -1e30
BIGI = 2**31 - 1


def _pass1_body(a_ref, w_ref, b_ref, l_ref, m_ref, s_ref, mrun_ref, srun_ref):
    c = pl.program_id(0)
    l = lax.dot_general(a_ref[...], w_ref[...],
                        dimension_numbers=(((1,), (1,)), ((), ())),
                        preferred_element_type=jnp.float32)
    l = l + b_ref[0]
    col = lax.broadcasted_iota(jnp.int32, (B, CC), 1) + c * CC
    l = jnp.where(col < C, l, NEG)
    l_ref[...] = l

    @pl.when(c == 0)
    def _():
        mrun_ref[...] = jnp.full((B, 1), NEG, jnp.float32)
        srun_ref[...] = jnp.zeros((B, 1), jnp.float32)

    mc = jnp.max(l, axis=1, keepdims=True)
    mprev = mrun_ref[...]
    mnew = jnp.maximum(mprev, mc)
    snew = (srun_ref[...] * jnp.exp(mprev - mnew)
            + jnp.sum(jnp.exp(l - mnew), axis=1, keepdims=True))
    mrun_ref[...] = mnew
    srun_ref[...] = snew
    m_ref[...] = jnp.broadcast_to(mnew, (B, 128))
    s_ref[...] = jnp.broadcast_to(snew, (B, 128))


_pass1 = pl.pallas_call(
    _pass1_body,
    grid=(NCH,),
    in_specs=[
        pl.BlockSpec((B, F), lambda c: (0, 0)),
        pl.BlockSpec((CC, F), lambda c: (c, 0)),
        pl.BlockSpec((1, 1, CC), lambda c: (c, 0, 0)),
    ],
    out_specs=[
        pl.BlockSpec((B, CC), lambda c: (0, c)),
        pl.BlockSpec((B, 128), lambda c: (0, 0)),
        pl.BlockSpec((B, 128), lambda c: (0, 0)),
    ],
    out_shape=[
        jax.ShapeDtypeStruct((B, CPAD), jnp.float32),
        jax.ShapeDtypeStruct((B, 128), jnp.float32),
        jax.ShapeDtypeStruct((B, 128), jnp.float32),
    ],
    scratch_shapes=[
        pltpu.VMEM((B, 1), jnp.float32),
        pltpu.VMEM((B, 1), jnp.float32),
    ],
)


def _pass2_body(l_ref, m_ref, s_ref, t_ref, cv_ref, ci_ref):
    c = pl.program_id(0)
    l = l_ref[...]
    m = jnp.max(m_ref[...], axis=1, keepdims=True)
    r = 1.0 / jnp.max(s_ref[...], axis=1, keepdims=True)
    t_ref[...] = jnp.exp(l - m) * r
    col = lax.broadcasted_iota(jnp.int32, (B, CC), 1) + c * CC
    lw = l
    vals, idxs = [], []
    for _ in range(K):
        mk = jnp.max(lw, axis=1, keepdims=True)
        ak = jnp.min(jnp.where(lw >= mk, col, BIGI), axis=1, keepdims=True)
        vals.append(mk)
        idxs.append(ak)
        lw = jnp.where(col == ak, NEG, lw)
    cv = jnp.concatenate(vals + [jnp.full((B, KSLOT - K), NEG, jnp.float32)], axis=1)
    ci = jnp.concatenate(idxs + [jnp.zeros((B, KSLOT - K), jnp.int32)], axis=1)
    cv_ref[...] = cv.reshape(1, B, KSLOT)
    ci_ref[...] = ci.reshape(1, B, KSLOT)


_pass2 = pl.pallas_call(
    _pass2_body,
    grid=(NCH,),
    in_specs=[
        pl.BlockSpec((B, CC), lambda c: (0, c)),
        pl.BlockSpec((B, 128), lambda c: (0, 0)),
        pl.BlockSpec((B, 128), lambda c: (0, 0)),
    ],
    out_specs=[
        pl.BlockSpec((B, CC), lambda c: (0, c)),
        pl.BlockSpec((1, B, KSLOT), lambda c: (c, 0, 0)),
        pl.BlockSpec((1, B, KSLOT), lambda c: (c, 0, 0)),
    ],
    out_shape=[
        jax.ShapeDtypeStruct((B, C), jnp.float32),
        jax.ShapeDtypeStruct((NCH, B, KSLOT), jnp.float32),
        jax.ShapeDtypeStruct((NCH, B, KSLOT), jnp.int32),
    ],
)


def _merge_body(cv_ref, ci_ref, topk_ref):
    ncand = NCH * KSLOT
    lane = lax.broadcasted_iota(jnp.int32, (B, ncand), 1)
    v = cv_ref[...]
    ci = ci_ref[...]
    idxs = []
    for _ in range(K):
        mk = jnp.max(v, axis=1, keepdims=True)
        pos = jnp.min(jnp.where(v >= mk, lane, BIGI), axis=1, keepdims=True)
        sel = lane == pos
        idxs.append(jnp.max(jnp.where(sel, ci, -1), axis=1, keepdims=True))
        v = jnp.where(sel, NEG, v)
    topk_ref[...] = jnp.concatenate(
        idxs + [jnp.zeros((B, KSLOT - K), jnp.int32)], axis=1)


_merge = pl.pallas_call(
    _merge_body,
    out_shape=jax.ShapeDtypeStruct((B, KSLOT), jnp.int32),
)


_NW = 32
_RPW = (B * K) // _NW
_GCH = 160


@functools.partial(
    pl.kernel,
    mesh=plsc.VectorSubcoreMesh(core_axis_name="c", subcore_axis_name="s"),
    out_type=jax.ShapeDtypeStruct((B * K, D), jnp.float32),
    scratch_types=[
        pltpu.VMEM((_RPW,), jnp.int32),
        pltpu.VMEM((_GCH, D), jnp.float32),
        pltpu.SemaphoreType.DMA,
    ],
)
def _sc_gather(idx_hbm, table_hbm, out_hbm, idx_v, rows_v, sem):
    wid = lax.axis_index("s") * 2 + lax.axis_index("c")
    base = wid * _RPW
    pltpu.sync_copy(idx_hbm.at[pl.ds(base, _RPW)], idx_v)
    for j in range(_RPW // _GCH):
        pltpu.async_copy(
            table_hbm.at[idx_v.at[pl.ds(j * _GCH, _GCH)]], rows_v, sem).wait()
        pltpu.sync_copy(rows_v, out_hbm.at[pl.ds(base + j * _GCH, _GCH)])


def kernel(avg_features, W, b, embed_table):
    b3 = jnp.pad(b, (0, CPAD - C)).reshape(NCH, 1, CC)
    lbuf, m, s = _pass1(avg_features, W, b3)
    tags, cv, ci = _pass2(lbuf, m, s)
    cvt = jnp.transpose(cv, (1, 0, 2)).reshape(B, NCH * KSLOT)
    cit = jnp.transpose(ci, (1, 0, 2)).reshape(B, NCH * KSLOT)
    topk = _merge(cvt, cit)
    idx_flat = topk[:, :K].reshape(B * K)
    sem_feat = _sc_gather(idx_flat, embed_table).reshape(B, K, D)
    return (tags, sem_feat)


# speedup vs baseline: 1.0346x; 1.0346x over previous
import functools

import jax
import jax.numpy as jnp
from jax import lax
from jax.experimental import pallas as pl
from jax.experimental.pallas import tpu as pltpu
from jax.experimental.pallas import tpu_sc as plsc

B = 1024
F = 2048
C = 100000
D = 512
K = 10
CC = 1024
NCH = 98
CC2 = 2048
NCH2 = 49
CPAD = NCH * CC
KSLOT = 16

NEG = -1e30
BIGI = 2**31 - 1


def _pass1_body(a_ref, w_ref, b_ref, l_ref, m_ref, s_ref, mrun_ref, srun_ref):
    c = pl.program_id(0)
    l = lax.dot_general(a_ref[...], w_ref[...],
                        dimension_numbers=(((1,), (1,)), ((), ())),
                        preferred_element_type=jnp.float32)
    l = l + b_ref[0]
    col = lax.broadcasted_iota(jnp.int32, (B, CC), 1) + c * CC
    l = jnp.where(col < C, l, NEG)
    l_ref[...] = l

    @pl.when(c == 0)
    def _():
        mrun_ref[...] = jnp.full((B, 1), NEG, jnp.float32)
        srun_ref[...] = jnp.zeros((B, 1), jnp.float32)

    mc = jnp.max(l, axis=1, keepdims=True)
    mprev = mrun_ref[...]
    mnew = jnp.maximum(mprev, mc)
    snew = (srun_ref[...] * jnp.exp(mprev - mnew)
            + jnp.sum(jnp.exp(l - mnew), axis=1, keepdims=True))
    mrun_ref[...] = mnew
    srun_ref[...] = snew
    m_ref[...] = jnp.broadcast_to(mnew, (B, 128))
    s_ref[...] = jnp.broadcast_to(snew, (B, 128))


_pass1 = pl.pallas_call(
    _pass1_body,
    grid=(NCH,),
    in_specs=[
        pl.BlockSpec((B, F), lambda c: (0, 0)),
        pl.BlockSpec((CC, F), lambda c: (c, 0)),
        pl.BlockSpec((1, 1, CC), lambda c: (c, 0, 0)),
    ],
    out_specs=[
        pl.BlockSpec((B, CC), lambda c: (0, c)),
        pl.BlockSpec((B, 128), lambda c: (0, 0)),
        pl.BlockSpec((B, 128), lambda c: (0, 0)),
    ],
    out_shape=[
        jax.ShapeDtypeStruct((B, CPAD), jnp.float32),
        jax.ShapeDtypeStruct((B, 128), jnp.float32),
        jax.ShapeDtypeStruct((B, 128), jnp.float32),
    ],
    scratch_shapes=[
        pltpu.VMEM((B, 1), jnp.float32),
        pltpu.VMEM((B, 1), jnp.float32),
    ],
)


def _pass2_body(l_ref, m_ref, s_ref, t_ref, cv_ref, ci_ref):
    c = pl.program_id(0)
    l = l_ref[...]
    m = jnp.max(m_ref[...], axis=1, keepdims=True)
    r = 1.0 / jnp.max(s_ref[...], axis=1, keepdims=True)
    t_ref[...] = jnp.exp(l - m) * r
    col = lax.broadcasted_iota(jnp.int32, (B, CC2), 1) + c * CC2
    lw = l
    vals, idxs = [], []
    for _ in range(K):
        mk = jnp.max(lw, axis=1, keepdims=True)
        ak = jnp.min(jnp.where(lw >= mk, col, BIGI), axis=1, keepdims=True)
        vals.append(mk)
        idxs.append(ak)
        lw = jnp.where(col == ak, NEG, lw)
    cv = jnp.concatenate(vals + [jnp.full((B, KSLOT - K), NEG, jnp.float32)], axis=1)
    ci = jnp.concatenate(idxs + [jnp.zeros((B, KSLOT - K), jnp.int32)], axis=1)
    cv_ref[...] = cv.reshape(1, B, KSLOT)
    ci_ref[...] = ci.reshape(1, B, KSLOT)


_pass2 = pl.pallas_call(
    _pass2_body,
    grid=(NCH2,),
    in_specs=[
        pl.BlockSpec((B, CC2), lambda c: (0, c)),
        pl.BlockSpec((B, 128), lambda c: (0, 0)),
        pl.BlockSpec((B, 128), lambda c: (0, 0)),
    ],
    out_specs=[
        pl.BlockSpec((B, CC2), lambda c: (0, c)),
        pl.BlockSpec((1, B, KSLOT), lambda c: (c, 0, 0)),
        pl.BlockSpec((1, B, KSLOT), lambda c: (c, 0, 0)),
    ],
    out_shape=[
        jax.ShapeDtypeStruct((B, C), jnp.float32),
        jax.ShapeDtypeStruct((NCH2, B, KSLOT), jnp.float32),
        jax.ShapeDtypeStruct((NCH2, B, KSLOT), jnp.int32),
    ],
)


def _merge_body(cv_ref, ci_ref, topk_ref):
    ncand = NCH2 * KSLOT
    lane = lax.broadcasted_iota(jnp.int32, (B, ncand), 1)
    v = cv_ref[...]
    ci = ci_ref[...]
    idxs = []
    for _ in range(K):
        mk = jnp.max(v, axis=1, keepdims=True)
        pos = jnp.min(jnp.where(v >= mk, lane, BIGI), axis=1, keepdims=True)
        sel = lane == pos
        idxs.append(jnp.max(jnp.where(sel, ci, -1), axis=1, keepdims=True))
        v = jnp.where(sel, NEG, v)
    topk_ref[...] = jnp.concatenate(
        idxs + [jnp.zeros((B, KSLOT - K), jnp.int32)], axis=1)


_merge = pl.pallas_call(
    _merge_body,
    out_shape=jax.ShapeDtypeStruct((B, KSLOT), jnp.int32),
)


_NW = 32
_RPW = (B * K) // _NW
_GCH = 160


@functools.partial(
    pl.kernel,
    mesh=plsc.VectorSubcoreMesh(core_axis_name="c", subcore_axis_name="s"),
    out_type=jax.ShapeDtypeStruct((B * K, D), jnp.float32),
    scratch_types=[
        pltpu.VMEM((_RPW,), jnp.int32),
        pltpu.VMEM((_GCH, D), jnp.float32),
        pltpu.SemaphoreType.DMA,
    ],
)
def _sc_gather(idx_hbm, table_hbm, out_hbm, idx_v, rows_v, sem):
    wid = lax.axis_index("s") * 2 + lax.axis_index("c")
    base = wid * _RPW
    pltpu.sync_copy(idx_hbm.at[pl.ds(base, _RPW)], idx_v)
    for j in range(_RPW // _GCH):
        pltpu.async_copy(
            table_hbm.at[idx_v.at[pl.ds(j * _GCH, _GCH)]], rows_v, sem).wait()
        pltpu.sync_copy(rows_v, out_hbm.at[pl.ds(base + j * _GCH, _GCH)])


def kernel(avg_features, W, b, embed_table):
    b3 = jnp.pad(b, (0, CPAD - C)).reshape(NCH, 1, CC)
    lbuf, m, s = _pass1(avg_features, W, b3)
    tags, cv, ci = _pass2(lbuf, m, s)
    cvt = jnp.transpose(cv, (1, 0, 2)).reshape(B, NCH2 * KSLOT)
    cit = jnp.transpose(ci, (1, 0, 2)).reshape(B, NCH2 * KSLOT)
    topk = _merge(cvt, cit)
    idx_flat = topk[:, :K].reshape(B * K)
    sem_feat = _sc_gather(idx_flat, embed_table).reshape(B, K, D)
    return (tags, sem_feat)
